# baseline (speedup 1.0000x reference)
import functools

import jax
import jax.numpy as jnp
from jax import lax
from jax.experimental import pallas as pl
from jax.experimental.pallas import tpu as pltpu
from jax.experimental.pallas import tpu_sc as plsc

N = 10000
E = 320000
E2 = E // 2
FA = 128
FB = 16
H = 128
DEPTH = 3

NC = 2
NS = 16
NW = NC * NS
EPW = E2 // NW
CHG = 200
NCHG = EPW // CHG
EPS = E2 // NS
CHS = 200
NCHS = EPS // CHS

NPAD = 10240
HALF = NPAD // NC
DUMP = 8
RPT = HALF // NS
ER = E // 128

_MESH = plsc.VectorSubcoreMesh(core_axis_name="c", subcore_axis_name="s")


def _sc_gather_body(e0, table_hbm, idx_hbm, out_hbm, idx_v0, idx_v1, idx_v2,
                    rows_v0, rows_v1, rows_v2, isem, gsem, osem):
    wid = lax.axis_index("s") * NC + lax.axis_index("c")
    base = wid * EPW
    idx_v = [idx_v0, idx_v1, idx_v2]
    rows_v = [rows_v0, rows_v1, rows_v2]

    icopies, ocopies = [], []

    def start_idx(j):
        icopies.append(pltpu.async_copy(
            idx_hbm.at[pl.ds(e0 + base + j * CHG, CHG)], idx_v[j % 3], isem))

    start_idx(0)
    start_idx(1)
    for j in range(NCHG):
        b = j % 3
        if j + 2 < NCHG:
            start_idx(j + 2)
        icopies[j].wait()
        if j >= 3:
            ocopies[j - 3].wait()
        pltpu.async_copy(table_hbm.at[idx_v[b]], rows_v[b], gsem).wait()
        ocopies.append(pltpu.async_copy(
            rows_v[b], out_hbm.at[pl.ds(base + j * CHG, CHG)], osem))
    for j in (NCHG - 3, NCHG - 2, NCHG - 1):
        ocopies[j].wait()


def _sc_gather(table, idx, e0):
    fn = pl.kernel(
        functools.partial(_sc_gather_body, e0),
        out_type=jax.ShapeDtypeStruct((E2, H), jnp.float32),
        mesh=_MESH,
        scratch_types=[
            pltpu.VMEM((CHG,), jnp.int32),
            pltpu.VMEM((CHG,), jnp.int32),
            pltpu.VMEM((CHG,), jnp.int32),
            pltpu.VMEM((CHG, H), jnp.float32),
            pltpu.VMEM((CHG, H), jnp.float32),
            pltpu.VMEM((CHG, H), jnp.float32),
            pltpu.SemaphoreType.DMA,
            pltpu.SemaphoreType.DMA,
            pltpu.SemaphoreType.DMA,
        ],
        name=f"sc_gather_{e0}",
    )
    return fn(table, idx)


def _sc_scatter_body(e0, vals_hbm, idx_hbm, init_hbm, out_hbm,
                     idx_v0, idx_v1, idx_v2, vals_v0, vals_v1, vals_v2, acc_sh,
                     isem, vsem, ssem):
    c = lax.axis_index("c")
    s = lax.axis_index("s")
    edge_base = s * EPS
    idx_v = [idx_v0, idx_v1, idx_v2]
    vals_v = [vals_v0, vals_v1, vals_v2]

    pltpu.sync_copy(init_hbm.at[pl.ds(c * HALF + s * RPT, RPT)],
                    acc_sh.at[pl.ds(s * RPT, RPT)])
    plsc.subcore_barrier()

    icopies, vcopies, scopies = [], [], []

    def start_in(j):
        b = j % 3
        off = edge_base + j * CHS
        icopies.append(pltpu.async_copy(
            idx_hbm.at[pl.ds(c * E + e0 + off, CHS)], idx_v[b], isem))
        vcopies.append(pltpu.async_copy(
            vals_hbm.at[pl.ds(off, CHS)], vals_v[b], vsem))

    start_in(0)
    for j in range(NCHS):
        b = j % 3
        if j >= 2:
            scopies[j - 2].wait()
        if j + 1 < NCHS:
            start_in(j + 1)
        icopies[j].wait()
        vcopies[j].wait()
        scopies.append(pltpu.async_copy(
            vals_v[b], acc_sh.at[idx_v[b]], ssem, add=True))
    scopies[NCHS - 2].wait()
    scopies[NCHS - 1].wait()
    plsc.subcore_barrier()
    pltpu.sync_copy(
        acc_sh.at[pl.ds(s * RPT, RPT)],
        out_hbm.at[pl.ds(c * HALF + s * RPT, RPT)],
    )


def _sc_scatter(vals, idx2, init, e0):
    fn = pl.kernel(
        functools.partial(_sc_scatter_body, e0),
        out_type=jax.ShapeDtypeStruct((NPAD, H), jnp.float32),
        mesh=_MESH,
        scratch_types=[
            pltpu.VMEM((CHS,), jnp.int32),
            pltpu.VMEM((CHS,), jnp.int32),
            pltpu.VMEM((CHS,), jnp.int32),
            pltpu.VMEM((CHS, H), jnp.float32),
            pltpu.VMEM((CHS, H), jnp.float32),
            pltpu.VMEM((CHS, H), jnp.float32),
            pltpu.VMEM_SHARED((HALF + DUMP, H), jnp.float32),
            pltpu.SemaphoreType.DMA,
            pltpu.SemaphoreType.DMA,
            pltpu.SemaphoreType.DMA,
        ],
        name=f"sc_scatter_{e0}",
    )
    return fn(vals, idx2, init)


def _remap_body(d_ref, o_ref):
    v = d_ref[...]
    for c in range(NC):
        lv = v - c * HALF
        ok = (lv >= 0) & (lv < HALF)
        o_ref[c] = jnp.where(ok, lv, HALF)


def _tc_remap(dst):
    out = pl.pallas_call(
        _remap_body,
        out_shape=jax.ShapeDtypeStruct((NC, ER, 128), jnp.int32),
    )(dst.reshape(ER, 128))
    return out.reshape(NC * E)


def _mm_body(x_ref, w_ref, o_ref):
    o_ref[...] = jnp.dot(x_ref[...], w_ref[...], preferred_element_type=jnp.float32)


def _tc_matmul(x, w):
    return pl.pallas_call(
        _mm_body,
        out_shape=jax.ShapeDtypeStruct((x.shape[0], w.shape[1]), jnp.float32),
    )(x, w)


BE = 2000
GRID_E2 = E2 // BE


def _init_body(ag_ref, bond_ref, wb_ref, b_ref, o_ref):
    acc = jnp.dot(bond_ref[...], wb_ref[...], preferred_element_type=jnp.float32)
    o_ref[...] = jnp.maximum(acc + ag_ref[...] + b_ref[...], 0.0)


def _tc_init(ag_half, bond, wb, b, half):
    return pl.pallas_call(
        _init_body,
        grid=(GRID_E2,),
        in_specs=[
            pl.BlockSpec((BE, H), lambda i: (i, 0)),
            pl.BlockSpec((BE, FB), lambda i, h=half: (i + h * GRID_E2, 0)),
            pl.BlockSpec((FB, H), lambda i: (0, 0)),
            pl.BlockSpec((1, H), lambda i: (0, 0)),
        ],
        out_specs=pl.BlockSpec((BE, H), lambda i: (i, 0)),
        out_shape=jax.ShapeDtypeStruct((E2, H), jnp.float32),
    )(ag_half, bond, wb, b)


def _combine_body(h_ref, m1_ref, w1_ref, w2_ref, b_ref, o_ref):
    hb = h_ref[...]
    up = pltpu.roll(hb, BE - 1, 0)
    down = pltpu.roll(hb, 1, 0)
    row = lax.broadcasted_iota(jnp.int32, hb.shape, 0)
    hrev = jnp.where((row & 1) == 0, up, down)
    m = m1_ref[...] - hrev
    acc = jnp.dot(hb, w1_ref[...], preferred_element_type=jnp.float32)
    acc = acc + jnp.dot(m, w2_ref[...], preferred_element_type=jnp.float32)
    o_ref[...] = jnp.maximum(acc + b_ref[...], 0.0)


def _tc_combine(h_half, m1_half, w1, w2, b):
    return pl.pallas_call(
        _combine_body,
        grid=(GRID_E2,),
        in_specs=[
            pl.BlockSpec((BE, H), lambda i: (i, 0)),
            pl.BlockSpec((BE, H), lambda i: (i, 0)),
            pl.BlockSpec((H, H), lambda i: (0, 0)),
            pl.BlockSpec((H, H), lambda i: (0, 0)),
            pl.BlockSpec((1, H), lambda i: (0, 0)),
        ],
        out_specs=pl.BlockSpec((BE, H), lambda i: (i, 0)),
        out_shape=jax.ShapeDtypeStruct((E2, H), jnp.float32),
    )(h_half, m1_half, w1, w2, b)


def _final_body(x_ref, w_ref, b_ref, o_ref):
    x = x_ref[...]
    ae = jnp.dot(x, w_ref[...], preferred_element_type=jnp.float32)
    ae = jnp.maximum(ae + b_ref[...], 0.0)
    row = lax.broadcasted_iota(jnp.int32, ae.shape, 0)
    ae = jnp.where(row < N, ae, 0.0)
    o_ref[...] = jnp.sum(ae, axis=0, keepdims=True)


def _tc_final(sum_in, w, b):
    return pl.pallas_call(
        _final_body,
        out_shape=jax.ShapeDtypeStruct((1, H), jnp.float32),
    )(sum_in, w, b)


def kernel(atom_features, bond_features, edge_index, W_ei, b_ei, W_eu, b_eu, W_nr, b_nr):
    src = edge_index[0]
    dst = edge_index[1]
    w_a = W_ei[:FA]
    w_b = W_ei[FA:]
    w_u1 = W_eu[:H]
    w_u2 = W_eu[H:]
    b_ei2 = b_ei.reshape(1, H)
    b_eu2 = b_eu.reshape(1, H)
    b_nr2 = b_nr.reshape(1, H)
    zeros_tab = jnp.zeros((NPAD, H), jnp.float32)

    idx2 = _tc_remap(dst)
    a = _tc_matmul(atom_features, w_a)
    ag_a = _sc_gather(a, src, 0)
    ag_b = _sc_gather(a, src, E2)
    h_a = _tc_init(ag_a, bond_features, w_b, b_ei2, 0)
    h_b = _tc_init(ag_b, bond_features, w_b, b_ei2, 1)

    for _ in range(DEPTH):
        s1 = _sc_scatter(h_a, idx2, zeros_tab, 0)
        sum_in = _sc_scatter(h_b, idx2, s1, E2)
        m1_a = _sc_gather(sum_in, src, 0)
        m1_b = _sc_gather(sum_in, src, E2)
        h_a = _tc_combine(h_a, m1_a, w_u1, w_u2, b_eu2)
        h_b = _tc_combine(h_b, m1_b, w_u1, w_u2, b_eu2)

    s1 = _sc_scatter(h_a, idx2, zeros_tab, 0)
    sum_in = _sc_scatter(h_b, idx2, s1, E2)
    out = _tc_final(sum_in, W_nr, b_nr2)
    return out.reshape(H)


# speedup vs baseline: 1.0795x; 1.0795x over previous
import functools

import jax
import jax.numpy as jnp
from jax import lax
from jax.experimental import pallas as pl
from jax.experimental.pallas import tpu as pltpu
from jax.experimental.pallas import tpu_sc as plsc

N = 10000
E = 320000
E2 = E // 2
FA = 128
FB = 16
H = 128
DEPTH = 3

NC = 2
NS = 16
NW = NC * NS
EPW = E2 // NW
CHG = 200
NCHG = EPW // CHG
EPS = E2 // NS
CHS = 200
NCHS = EPS // CHS

NPAD = 10240
HALF = NPAD // NC
DUMP = 8
RPT = HALF // NS
ER = E // 128

_MESH = plsc.VectorSubcoreMesh(core_axis_name="c", subcore_axis_name="s")


CHGT = 40
NCHGT = EPW // CHGT
TLOAD = 632


def _sc_gather_body(e0, table_hbm, idx_hbm, out_hbm, idx_v0, idx_v1, idx_v2,
                    rows_v0, rows_v1, rows_v2, tab_sh, isem, gsem, osem):
    c = lax.axis_index("c")
    s = lax.axis_index("s")
    wid = s * NC + c
    base = wid * EPW
    idx_v = [idx_v0, idx_v1, idx_v2]
    rows_v = [rows_v0, rows_v1, rows_v2]

    @pl.when(s < NS - 1)
    def _():
        pltpu.sync_copy(table_hbm.at[pl.ds(s * TLOAD, TLOAD)],
                        tab_sh.at[pl.ds(s * TLOAD, TLOAD)])

    @pl.when(s == NS - 1)
    def _():
        pltpu.sync_copy(table_hbm.at[pl.ds((NS - 1) * TLOAD, N - (NS - 1) * TLOAD)],
                        tab_sh.at[pl.ds((NS - 1) * TLOAD, N - (NS - 1) * TLOAD)])

    plsc.subcore_barrier()

    icopies, ocopies = [], []

    def start_idx(j):
        icopies.append(pltpu.async_copy(
            idx_hbm.at[pl.ds(e0 + base + j * CHGT, CHGT)], idx_v[j % 3], isem))

    start_idx(0)
    start_idx(1)
    for j in range(NCHGT):
        b = j % 3
        if j + 2 < NCHGT:
            start_idx(j + 2)
        icopies[j].wait()
        if j >= 3:
            ocopies[j - 3].wait()
        pltpu.async_copy(tab_sh.at[idx_v[b]], rows_v[b], gsem).wait()
        ocopies.append(pltpu.async_copy(
            rows_v[b], out_hbm.at[pl.ds(base + j * CHGT, CHGT)], osem))
    for j in (NCHGT - 3, NCHGT - 2, NCHGT - 1):
        ocopies[j].wait()


def _sc_gather(table, idx, e0):
    fn = pl.kernel(
        functools.partial(_sc_gather_body, e0),
        out_type=jax.ShapeDtypeStruct((E2, H), jnp.float32),
        mesh=_MESH,
        scratch_types=[
            pltpu.VMEM((CHGT,), jnp.int32),
            pltpu.VMEM((CHGT,), jnp.int32),
            pltpu.VMEM((CHGT,), jnp.int32),
            pltpu.VMEM((CHGT, H), jnp.float32),
            pltpu.VMEM((CHGT, H), jnp.float32),
            pltpu.VMEM((CHGT, H), jnp.float32),
            pltpu.VMEM_SHARED((N, H), jnp.float32),
            pltpu.SemaphoreType.DMA,
            pltpu.SemaphoreType.DMA,
            pltpu.SemaphoreType.DMA,
        ],
        name=f"sc_gather_{e0}",
    )
    return fn(table, idx)


def _sc_scatter_body(e0, vals_hbm, idx_hbm, init_hbm, out_hbm,
                     idx_v0, idx_v1, idx_v2, vals_v0, vals_v1, vals_v2, acc_sh,
                     isem, vsem, ssem):
    c = lax.axis_index("c")
    s = lax.axis_index("s")
    edge_base = s * EPS
    idx_v = [idx_v0, idx_v1, idx_v2]
    vals_v = [vals_v0, vals_v1, vals_v2]

    pltpu.sync_copy(init_hbm.at[pl.ds(c * HALF + s * RPT, RPT)],
                    acc_sh.at[pl.ds(s * RPT, RPT)])
    plsc.subcore_barrier()

    icopies, vcopies, scopies = [], [], []

    def start_in(j):
        b = j % 3
        off = edge_base + j * CHS
        icopies.append(pltpu.async_copy(
            idx_hbm.at[pl.ds(c * E + e0 + off, CHS)], idx_v[b], isem))
        vcopies.append(pltpu.async_copy(
            vals_hbm.at[pl.ds(off, CHS)], vals_v[b], vsem))

    start_in(0)
    for j in range(NCHS):
        b = j % 3
        if j >= 2:
            scopies[j - 2].wait()
        if j + 1 < NCHS:
            start_in(j + 1)
        icopies[j].wait()
        vcopies[j].wait()
        scopies.append(pltpu.async_copy(
            vals_v[b], acc_sh.at[idx_v[b]], ssem, add=True))
    scopies[NCHS - 2].wait()
    scopies[NCHS - 1].wait()
    plsc.subcore_barrier()
    pltpu.sync_copy(
        acc_sh.at[pl.ds(s * RPT, RPT)],
        out_hbm.at[pl.ds(c * HALF + s * RPT, RPT)],
    )


def _sc_scatter(vals, idx2, init, e0):
    fn = pl.kernel(
        functools.partial(_sc_scatter_body, e0),
        out_type=jax.ShapeDtypeStruct((NPAD, H), jnp.float32),
        mesh=_MESH,
        scratch_types=[
            pltpu.VMEM((CHS,), jnp.int32),
            pltpu.VMEM((CHS,), jnp.int32),
            pltpu.VMEM((CHS,), jnp.int32),
            pltpu.VMEM((CHS, H), jnp.float32),
            pltpu.VMEM((CHS, H), jnp.float32),
            pltpu.VMEM((CHS, H), jnp.float32),
            pltpu.VMEM_SHARED((HALF + DUMP, H), jnp.float32),
            pltpu.SemaphoreType.DMA,
            pltpu.SemaphoreType.DMA,
            pltpu.SemaphoreType.DMA,
        ],
        name=f"sc_scatter_{e0}",
    )
    return fn(vals, idx2, init)


def _remap_body(d_ref, o_ref):
    v = d_ref[...]
    for c in range(NC):
        lv = v - c * HALF
        ok = (lv >= 0) & (lv < HALF)
        o_ref[c] = jnp.where(ok, lv, HALF)


def _tc_remap(dst):
    out = pl.pallas_call(
        _remap_body,
        out_shape=jax.ShapeDtypeStruct((NC, ER, 128), jnp.int32),
    )(dst.reshape(ER, 128))
    return out.reshape(NC * E)


def _mm_body(x_ref, w_ref, o_ref):
    o_ref[...] = jnp.dot(x_ref[...], w_ref[...], preferred_element_type=jnp.float32)


def _tc_matmul(x, w):
    return pl.pallas_call(
        _mm_body,
        out_shape=jax.ShapeDtypeStruct((x.shape[0], w.shape[1]), jnp.float32),
    )(x, w)


BE = 2000
GRID_E2 = E2 // BE


def _init_body(ag_ref, bond_ref, wb_ref, b_ref, o_ref):
    acc = jnp.dot(bond_ref[...], wb_ref[...], preferred_element_type=jnp.float32)
    o_ref[...] = jnp.maximum(acc + ag_ref[...] + b_ref[...], 0.0)


def _tc_init(ag_half, bond, wb, b, half):
    return pl.pallas_call(
        _init_body,
        grid=(GRID_E2,),
        in_specs=[
            pl.BlockSpec((BE, H), lambda i: (i, 0)),
            pl.BlockSpec((BE, FB), lambda i, h=half: (i + h * GRID_E2, 0)),
            pl.BlockSpec((FB, H), lambda i: (0, 0)),
            pl.BlockSpec((1, H), lambda i: (0, 0)),
        ],
        out_specs=pl.BlockSpec((BE, H), lambda i: (i, 0)),
        out_shape=jax.ShapeDtypeStruct((E2, H), jnp.float32),
    )(ag_half, bond, wb, b)


def _combine_body(h_ref, m1_ref, w1_ref, w2_ref, b_ref, o_ref):
    hb = h_ref[...]
    up = pltpu.roll(hb, BE - 1, 0)
    down = pltpu.roll(hb, 1, 0)
    row = lax.broadcasted_iota(jnp.int32, hb.shape, 0)
    hrev = jnp.where((row & 1) == 0, up, down)
    m = m1_ref[...] - hrev
    acc = jnp.dot(hb, w1_ref[...], preferred_element_type=jnp.float32)
    acc = acc + jnp.dot(m, w2_ref[...], preferred_element_type=jnp.float32)
    o_ref[...] = jnp.maximum(acc + b_ref[...], 0.0)


def _tc_combine(h_half, m1_half, w1, w2, b):
    return pl.pallas_call(
        _combine_body,
        grid=(GRID_E2,),
        in_specs=[
            pl.BlockSpec((BE, H), lambda i: (i, 0)),
            pl.BlockSpec((BE, H), lambda i: (i, 0)),
            pl.BlockSpec((H, H), lambda i: (0, 0)),
            pl.BlockSpec((H, H), lambda i: (0, 0)),
            pl.BlockSpec((1, H), lambda i: (0, 0)),
        ],
        out_specs=pl.BlockSpec((BE, H), lambda i: (i, 0)),
        out_shape=jax.ShapeDtypeStruct((E2, H), jnp.float32),
    )(h_half, m1_half, w1, w2, b)


def _final_body(x_ref, w_ref, b_ref, o_ref):
    x = x_ref[...]
    ae = jnp.dot(x, w_ref[...], preferred_element_type=jnp.float32)
    ae = jnp.maximum(ae + b_ref[...], 0.0)
    row = lax.broadcasted_iota(jnp.int32, ae.shape, 0)
    ae = jnp.where(row < N, ae, 0.0)
    o_ref[...] = jnp.sum(ae, axis=0, keepdims=True)


def _tc_final(sum_in, w, b):
    return pl.pallas_call(
        _final_body,
        out_shape=jax.ShapeDtypeStruct((1, H), jnp.float32),
    )(sum_in, w, b)


def kernel(atom_features, bond_features, edge_index, W_ei, b_ei, W_eu, b_eu, W_nr, b_nr):
    src = edge_index[0]
    dst = edge_index[1]
    w_a = W_ei[:FA]
    w_b = W_ei[FA:]
    w_u1 = W_eu[:H]
    w_u2 = W_eu[H:]
    b_ei2 = b_ei.reshape(1, H)
    b_eu2 = b_eu.reshape(1, H)
    b_nr2 = b_nr.reshape(1, H)
    zeros_tab = jnp.zeros((NPAD, H), jnp.float32)

    idx2 = _tc_remap(dst)
    a = _tc_matmul(atom_features, w_a)
    ag_a = _sc_gather(a, src, 0)
    ag_b = _sc_gather(a, src, E2)
    h_a = _tc_init(ag_a, bond_features, w_b, b_ei2, 0)
    h_b = _tc_init(ag_b, bond_features, w_b, b_ei2, 1)

    for _ in range(DEPTH):
        s1 = _sc_scatter(h_a, idx2, zeros_tab, 0)
        sum_in = _sc_scatter(h_b, idx2, s1, E2)
        m1_a = _sc_gather(sum_in, src, 0)
        m1_b = _sc_gather(sum_in, src, E2)
        h_a = _tc_combine(h_a, m1_a, w_u1, w_u2, b_eu2)
        h_b = _tc_combine(h_b, m1_b, w_u1, w_u2, b_eu2)

    s1 = _sc_scatter(h_a, idx2, zeros_tab, 0)
    sum_in = _sc_scatter(h_b, idx2, s1, E2)
    out = _tc_final(sum_in, W_nr, b_nr2)
    return out.reshape(H)


# speedup vs baseline: 1.2853x; 1.1906x over previous
import functools

import jax
import jax.numpy as jnp
from jax import lax
from jax.experimental import pallas as pl
from jax.experimental.pallas import tpu as pltpu
from jax.experimental.pallas import tpu_sc as plsc

N = 10000
E = 320000
E2 = E // 2
FA = 128
FB = 16
H = 128
DEPTH = 3

NC = 2
NS = 16
NW = NC * NS
EPW = E2 // NW
CHG = 200
NCHG = EPW // CHG
EH = E2 // NC
EPS = EH // NS
CHS = 40
NCHS = EPS // CHS

NPAD = 10240
RPT = NPAD // NS

_MESH = plsc.VectorSubcoreMesh(core_axis_name="c", subcore_axis_name="s")


CHGT = 40
NCHGT = EPW // CHGT
TLOAD = 632


def _sc_gather_body(e0, table_hbm, idx_hbm, out_hbm, idx_v0, idx_v1, idx_v2,
                    rows_v0, rows_v1, rows_v2, tab_sh, isem, gsem, osem):
    c = lax.axis_index("c")
    s = lax.axis_index("s")
    wid = s * NC + c
    base = wid * EPW
    idx_v = [idx_v0, idx_v1, idx_v2]
    rows_v = [rows_v0, rows_v1, rows_v2]

    @pl.when(s < NS - 1)
    def _():
        pltpu.sync_copy(table_hbm.at[pl.ds(s * TLOAD, TLOAD)],
                        tab_sh.at[pl.ds(s * TLOAD, TLOAD)])

    @pl.when(s == NS - 1)
    def _():
        pltpu.sync_copy(table_hbm.at[pl.ds((NS - 1) * TLOAD, N - (NS - 1) * TLOAD)],
                        tab_sh.at[pl.ds((NS - 1) * TLOAD, N - (NS - 1) * TLOAD)])

    plsc.subcore_barrier()

    icopies, ocopies = [], []

    def start_idx(j):
        icopies.append(pltpu.async_copy(
            idx_hbm.at[pl.ds(e0 + base + j * CHGT, CHGT)], idx_v[j % 3], isem))

    start_idx(0)
    start_idx(1)
    for j in range(NCHGT):
        b = j % 3
        if j + 2 < NCHGT:
            start_idx(j + 2)
        icopies[j].wait()
        if j >= 3:
            ocopies[j - 3].wait()
        pltpu.async_copy(tab_sh.at[idx_v[b]], rows_v[b], gsem).wait()
        ocopies.append(pltpu.async_copy(
            rows_v[b], out_hbm.at[pl.ds(base + j * CHGT, CHGT)], osem))
    for j in (NCHGT - 3, NCHGT - 2, NCHGT - 1):
        ocopies[j].wait()


def _sc_gather(table, idx, e0):
    fn = pl.kernel(
        functools.partial(_sc_gather_body, e0),
        out_type=jax.ShapeDtypeStruct((E2, H), jnp.float32),
        mesh=_MESH,
        scratch_types=[
            pltpu.VMEM((CHGT,), jnp.int32),
            pltpu.VMEM((CHGT,), jnp.int32),
            pltpu.VMEM((CHGT,), jnp.int32),
            pltpu.VMEM((CHGT, H), jnp.float32),
            pltpu.VMEM((CHGT, H), jnp.float32),
            pltpu.VMEM((CHGT, H), jnp.float32),
            pltpu.VMEM_SHARED((N, H), jnp.float32),
            pltpu.SemaphoreType.DMA,
            pltpu.SemaphoreType.DMA,
            pltpu.SemaphoreType.DMA,
        ],
        name=f"sc_gather_{e0}",
    )
    return fn(table, idx)


def _sc_scatter_body(e0, vals_hbm, idx_hbm, init_hbm, out_hbm,
                     idx_v0, idx_v1, idx_v2, vals_v0, vals_v1, vals_v2, acc_sh,
                     isem, vsem, ssem):
    c = lax.axis_index("c")
    s = lax.axis_index("s")
    edge_base = c * EH + s * EPS
    idx_v = [idx_v0, idx_v1, idx_v2]
    vals_v = [vals_v0, vals_v1, vals_v2]

    pltpu.sync_copy(init_hbm.at[pl.ds(c * NPAD + s * RPT, RPT)],
                    acc_sh.at[pl.ds(s * RPT, RPT)])
    plsc.subcore_barrier()

    icopies, vcopies, scopies = [], [], []

    def start_in(j):
        b = j % 3
        off = edge_base + j * CHS
        icopies.append(pltpu.async_copy(
            idx_hbm.at[pl.ds(e0 + off, CHS)], idx_v[b], isem))
        vcopies.append(pltpu.async_copy(
            vals_hbm.at[pl.ds(off, CHS)], vals_v[b], vsem))

    start_in(0)
    for j in range(NCHS):
        b = j % 3
        if j >= 2:
            scopies[j - 2].wait()
        if j + 1 < NCHS:
            start_in(j + 1)
        icopies[j].wait()
        vcopies[j].wait()
        scopies.append(pltpu.async_copy(
            vals_v[b], acc_sh.at[idx_v[b]], ssem, add=True))
    scopies[NCHS - 2].wait()
    scopies[NCHS - 1].wait()
    plsc.subcore_barrier()
    pltpu.sync_copy(
        acc_sh.at[pl.ds(s * RPT, RPT)],
        out_hbm.at[pl.ds(c * NPAD + s * RPT, RPT)],
    )


def _sc_scatter(vals, idx, init, e0):
    fn = pl.kernel(
        functools.partial(_sc_scatter_body, e0),
        out_type=jax.ShapeDtypeStruct((NC * NPAD, H), jnp.float32),
        mesh=_MESH,
        scratch_types=[
            pltpu.VMEM((CHS,), jnp.int32),
            pltpu.VMEM((CHS,), jnp.int32),
            pltpu.VMEM((CHS,), jnp.int32),
            pltpu.VMEM((CHS, H), jnp.float32),
            pltpu.VMEM((CHS, H), jnp.float32),
            pltpu.VMEM((CHS, H), jnp.float32),
            pltpu.VMEM_SHARED((NPAD, H), jnp.float32),
            pltpu.SemaphoreType.DMA,
            pltpu.SemaphoreType.DMA,
            pltpu.SemaphoreType.DMA,
        ],
        name=f"sc_scatter_{e0}",
    )
    return fn(vals, idx, init)


def _addparts_body(p_ref, o_ref):
    o_ref[...] = p_ref[:NPAD] + p_ref[NPAD:]


def _tc_addparts(parts):
    return pl.pallas_call(
        _addparts_body,
        out_shape=jax.ShapeDtypeStruct((NPAD, H), jnp.float32),
    )(parts)


def _mm_body(x_ref, w_ref, o_ref):
    o_ref[...] = jnp.dot(x_ref[...], w_ref[...], preferred_element_type=jnp.float32)


def _tc_matmul(x, w):
    return pl.pallas_call(
        _mm_body,
        out_shape=jax.ShapeDtypeStruct((x.shape[0], w.shape[1]), jnp.float32),
    )(x, w)


BE = 2000
GRID_E2 = E2 // BE


def _init_body(ag_ref, bond_ref, wb_ref, b_ref, o_ref):
    acc = jnp.dot(bond_ref[...], wb_ref[...], preferred_element_type=jnp.float32)
    o_ref[...] = jnp.maximum(acc + ag_ref[...] + b_ref[...], 0.0)


def _tc_init(ag_half, bond, wb, b, half):
    return pl.pallas_call(
        _init_body,
        grid=(GRID_E2,),
        in_specs=[
            pl.BlockSpec((BE, H), lambda i: (i, 0)),
            pl.BlockSpec((BE, FB), lambda i, h=half: (i + h * GRID_E2, 0)),
            pl.BlockSpec((FB, H), lambda i: (0, 0)),
            pl.BlockSpec((1, H), lambda i: (0, 0)),
        ],
        out_specs=pl.BlockSpec((BE, H), lambda i: (i, 0)),
        out_shape=jax.ShapeDtypeStruct((E2, H), jnp.float32),
    )(ag_half, bond, wb, b)


def _combine_body(h_ref, m1_ref, w1_ref, w2_ref, b_ref, o_ref):
    hb = h_ref[...]
    up = pltpu.roll(hb, BE - 1, 0)
    down = pltpu.roll(hb, 1, 0)
    row = lax.broadcasted_iota(jnp.int32, hb.shape, 0)
    hrev = jnp.where((row & 1) == 0, up, down)
    m = m1_ref[...] - hrev
    acc = jnp.dot(hb, w1_ref[...], preferred_element_type=jnp.float32)
    acc = acc + jnp.dot(m, w2_ref[...], preferred_element_type=jnp.float32)
    o_ref[...] = jnp.maximum(acc + b_ref[...], 0.0)


def _tc_combine(h_half, m1_half, w1, w2, b):
    return pl.pallas_call(
        _combine_body,
        grid=(GRID_E2,),
        in_specs=[
            pl.BlockSpec((BE, H), lambda i: (i, 0)),
            pl.BlockSpec((BE, H), lambda i: (i, 0)),
            pl.BlockSpec((H, H), lambda i: (0, 0)),
            pl.BlockSpec((H, H), lambda i: (0, 0)),
            pl.BlockSpec((1, H), lambda i: (0, 0)),
        ],
        out_specs=pl.BlockSpec((BE, H), lambda i: (i, 0)),
        out_shape=jax.ShapeDtypeStruct((E2, H), jnp.float32),
    )(h_half, m1_half, w1, w2, b)


def _final_body(p_ref, w_ref, b_ref, o_ref):
    x = p_ref[:NPAD] + p_ref[NPAD:]
    ae = jnp.dot(x, w_ref[...], preferred_element_type=jnp.float32)
    ae = jnp.maximum(ae + b_ref[...], 0.0)
    row = lax.broadcasted_iota(jnp.int32, ae.shape, 0)
    ae = jnp.where(row < N, ae, 0.0)
    o_ref[...] = jnp.sum(ae, axis=0, keepdims=True)


def _tc_final(parts, w, b):
    return pl.pallas_call(
        _final_body,
        out_shape=jax.ShapeDtypeStruct((1, H), jnp.float32),
    )(parts, w, b)


def kernel(atom_features, bond_features, edge_index, W_ei, b_ei, W_eu, b_eu, W_nr, b_nr):
    src = edge_index[0]
    dst = edge_index[1]
    w_a = W_ei[:FA]
    w_b = W_ei[FA:]
    w_u1 = W_eu[:H]
    w_u2 = W_eu[H:]
    b_ei2 = b_ei.reshape(1, H)
    b_eu2 = b_eu.reshape(1, H)
    b_nr2 = b_nr.reshape(1, H)
    zeros_tab = jnp.zeros((NC * NPAD, H), jnp.float32)

    a = _tc_matmul(atom_features, w_a)
    ag_a = _sc_gather(a, src, 0)
    ag_b = _sc_gather(a, src, E2)
    h_a = _tc_init(ag_a, bond_features, w_b, b_ei2, 0)
    h_b = _tc_init(ag_b, bond_features, w_b, b_ei2, 1)

    for _ in range(DEPTH):
        s1 = _sc_scatter(h_a, dst, zeros_tab, 0)
        s2 = _sc_scatter(h_b, dst, s1, E2)
        sum_in = _tc_addparts(s2)
        m1_a = _sc_gather(sum_in, src, 0)
        m1_b = _sc_gather(sum_in, src, E2)
        h_a = _tc_combine(h_a, m1_a, w_u1, w_u2, b_eu2)
        h_b = _tc_combine(h_b, m1_b, w_u1, w_u2, b_eu2)

    s1 = _sc_scatter(h_a, dst, zeros_tab, 0)
    s2 = _sc_scatter(h_b, dst, s1, E2)
    out = _tc_final(s2, W_nr, b_nr2)
    return out.reshape(H)
